# Initial kernel scaffold; baseline (speedup 1.0000x reference)
import jax
import jax.numpy as jnp
from jax.experimental import pallas as pl


def kernel(x, edge_index, W1, b1, W2, b2):
    raise NotImplementedError("write your pallas kernel here")


# speedup vs baseline: 13.6585x; 13.6585x over previous
import functools

import jax
import jax.numpy as jnp
from jax import lax
from jax.experimental import pallas as pl
from jax.experimental.pallas import tpu as pltpu
from jax.experimental.pallas import tpu_sc as plsc

N_NODES = 10000
D = 128
DH = D // 2
N_EDGES = 320000

NC = 2
NS = 16

NP = 10240
RPS = NP // NS
CH = 128
NCH = 160
E_PAD = NS * NCH * CH
PAD_DST = N_NODES

BLK = 1024


_mesh = plsc.VectorSubcoreMesh(core_axis_name="c", subcore_axis_name="s")


@functools.partial(
    pl.kernel,
    out_type=jax.ShapeDtypeStruct((NC, NP, 16), jnp.float32),
    mesh=_mesh,
    compiler_params=pltpu.CompilerParams(use_tc_tiling_on_sc=False),
    scratch_types=[
        pltpu.VMEM((NCH, CH), jnp.int32),
        pltpu.VMEM((CH, 16), jnp.float32),
        pltpu.VMEM_SHARED((NP, 16), jnp.float32),
    ],
)
def _deg_sc(dst_hbm, z16_hbm, ones_hbm, out_hbm, didx, ones_v, acc_sh):
    c = lax.axis_index("c")
    s = lax.axis_index("s")
    pltpu.sync_copy(dst_hbm.at[s], didx)
    pltpu.sync_copy(ones_hbm, ones_v)
    pltpu.sync_copy(z16_hbm.at[pl.ds(s * RPS, RPS)],
                    acc_sh.at[pl.ds(s * RPS, RPS)])
    plsc.subcore_barrier()

    @pl.loop(0, NCH)
    def _(j):
        pltpu.sync_copy(ones_v, acc_sh.at[didx.at[j]], add=True)

    plsc.subcore_barrier()
    pltpu.sync_copy(acc_sh.at[pl.ds(s * RPS, RPS)],
                    out_hbm.at[c, pl.ds(s * RPS, RPS)])


@functools.partial(
    pl.kernel,
    out_type=jax.ShapeDtypeStruct((NC, NP, DH), jnp.float32),
    mesh=_mesh,
    compiler_params=pltpu.CompilerParams(use_tc_tiling_on_sc=False),
    scratch_types=[
        pltpu.VMEM((NCH, CH), jnp.int32),
        pltpu.VMEM((NCH, CH), jnp.int32),
        pltpu.VMEM((CH, DH), jnp.float32),
        pltpu.VMEM((CH, DH), jnp.float32),
        pltpu.VMEM_SHARED((NP, DH), jnp.float32),
        pltpu.SemaphoreType.DMA,
        pltpu.SemaphoreType.DMA,
    ],
)
def _prop_sc(h_hbm, src_hbm, dst_hbm, zh_hbm, out_hbm,
             sidx, didx, rows0, rows1, acc_sh, sem0, sem1):
    c = lax.axis_index("c")
    s = lax.axis_index("s")
    hview = h_hbm.at[c]
    pltpu.sync_copy(src_hbm.at[s], sidx)
    pltpu.sync_copy(dst_hbm.at[s], didx)
    pltpu.sync_copy(zh_hbm.at[pl.ds(s * RPS, RPS)],
                    acc_sh.at[pl.ds(s * RPS, RPS)])
    plsc.subcore_barrier()

    pltpu.make_async_copy(hview.at[sidx.at[0]], rows0, sem0).start()
    pltpu.make_async_copy(hview.at[sidx.at[1]], rows1, sem1).start()

    @pl.loop(0, NCH, step=2)
    def _(j):
        pltpu.make_async_copy(hview.at[sidx.at[j]], rows0, sem0).wait()
        pltpu.sync_copy(rows0, acc_sh.at[didx.at[j]], add=True)

        @pl.when(j + 2 < NCH)
        def _():
            pltpu.make_async_copy(hview.at[sidx.at[j + 2]], rows0, sem0).start()

        pltpu.make_async_copy(hview.at[sidx.at[j + 1]], rows1, sem1).wait()
        pltpu.sync_copy(rows1, acc_sh.at[didx.at[j + 1]], add=True)

        @pl.when(j + 3 < NCH)
        def _():
            pltpu.make_async_copy(hview.at[sidx.at[j + 3]], rows1, sem1).start()

    plsc.subcore_barrier()
    pltpu.sync_copy(acc_sh.at[pl.ds(s * RPS, RPS)],
                    out_hbm.at[c, pl.ds(s * RPS, RPS)])


def _dinv_of(dg_ref):
    deg = dg_ref[:, 0:1] + 1.0
    return lax.rsqrt(deg)


def _split_store(o_ref, full):
    o_ref[0, :, :] = full[:, :DH]
    o_ref[1, :, :] = full[:, DH:]


def _cat(ref):
    return jnp.concatenate([ref[0], ref[1]], axis=1)


def _mm1_body(x_ref, w_ref, dg_ref, o_ref):
    dinv = _dinv_of(dg_ref)
    full = jnp.dot(x_ref[...], w_ref[...],
                   preferred_element_type=jnp.float32) * dinv
    _split_store(o_ref, full)


def _fuse2_body(a_ref, h_ref, dg_ref, b_ref, w_ref, o_ref):
    dinv = _dinv_of(dg_ref)
    z = dinv * (_cat(a_ref) + _cat(h_ref)) + b_ref[...]
    z = jnp.maximum(z, 0.0)
    full = jnp.dot(z, w_ref[...], preferred_element_type=jnp.float32) * dinv
    _split_store(o_ref, full)


def _out_body(a_ref, h_ref, dg_ref, b_ref, o_ref):
    dinv = _dinv_of(dg_ref)
    o_ref[...] = dinv * (_cat(a_ref) + _cat(h_ref)) + b_ref[...]


def _row_spec(width):
    return pl.BlockSpec((BLK, width), lambda i: (i, 0))


def _half_spec():
    return pl.BlockSpec((NC, BLK, DH), lambda i: (0, i, 0))


def _const_spec(shape):
    return pl.BlockSpec(shape, lambda i: (0, 0))


_HALves = jax.ShapeDtypeStruct((NC, NP, DH), jnp.float32)
_GRID = (NP // BLK,)

_mm1_tc = pl.pallas_call(
    _mm1_body,
    grid=_GRID,
    in_specs=[_row_spec(D), _const_spec((D, D)), _row_spec(16)],
    out_specs=_half_spec(),
    out_shape=_HALves,
)

_fuse2_tc = pl.pallas_call(
    _fuse2_body,
    grid=_GRID,
    in_specs=[_half_spec(), _half_spec(), _row_spec(16),
              _const_spec((1, D)), _const_spec((D, D))],
    out_specs=_half_spec(),
    out_shape=_HALves,
)

_out_tc = pl.pallas_call(
    _out_body,
    grid=_GRID,
    in_specs=[_half_spec(), _half_spec(), _row_spec(16), _const_spec((1, D))],
    out_specs=_row_spec(D),
    out_shape=jax.ShapeDtypeStruct((NP, D), jnp.float32),
)


def kernel(x, edge_index, W1, b1, W2, b2):
    ei = edge_index.astype(jnp.int32)
    pad = E_PAD - N_EDGES
    src = jnp.concatenate([ei[0], jnp.zeros((pad,), jnp.int32)])
    dst = jnp.concatenate([ei[1], jnp.full((pad,), PAD_DST, jnp.int32)])
    src_r = src.reshape(NS, NCH, CH)
    dst_r = dst.reshape(NS, NCH, CH)

    x_pad = jnp.pad(x, ((0, NP - N_NODES), (0, 0)))
    zh = jnp.zeros((NP, DH), jnp.float32)
    z16 = jnp.zeros((NP, 16), jnp.float32)
    ones16 = jnp.ones((CH, 16), jnp.float32)
    b1r = b1.reshape(1, D)
    b2r = b2.reshape(1, D)

    degw = _deg_sc(dst_r, z16, ones16)
    dg = degw[0]

    h1p = _mm1_tc(x_pad, W1, dg)
    acc1 = _prop_sc(h1p, src_r, dst_r, zh)
    h2p = _fuse2_tc(acc1, h1p, dg, b1r, W2)
    acc2 = _prop_sc(h2p, src_r, dst_r, zh)
    out = _out_tc(acc2, h2p, dg, b2r)
    return out[:N_NODES]
